# baseline (speedup 1.0000x reference)
import jax
import jax.numpy as jnp
from jax import lax
from jax.experimental import pallas as pl
from jax.experimental.pallas import tpu as pltpu
from jax.experimental.pallas import tpu_sc as plsc

D_MODEL = 768
N_EXPERTS = 64
N_TOKENS = 2048
ROW_TILE = 64
N_BUF = 2560
PAD = 8
TOK_PER_W = 64
RBLK = 128

_INTERPRET = False


N_RT = N_TOKENS // RBLK


def _router_body(x_ref, wr_ref, br_ref, logits_ref, w_ref, pos_ref, offs_ref,
                 oh_sc, cnt_sc, base_sc):
    i = pl.program_id(0)
    lane = lax.broadcasted_iota(jnp.int32, (RBLK, N_EXPERTS), 1)

    @pl.when(i < N_RT)
    def _pass1():
        logits = lax.dot_general(
            x_ref[...], wr_ref[...], (((1,), (1,)), ((), ())),
            preferred_element_type=jnp.float32) + br_ref[...]
        logits_ref[...] = logits
        m = jnp.max(logits, axis=1, keepdims=True)
        s = jnp.sum(jnp.exp(logits - m), axis=1, keepdims=True)
        w_ref[...] = 1.0 / s
        cand = jnp.where(logits == m, lane, N_EXPERTS)
        idx = jnp.min(cand, axis=1, keepdims=True)
        onehot = (lane == idx).astype(jnp.float32)
        oh_sc[pl.ds(i * RBLK, RBLK), :] = onehot
        csum = jnp.sum(onehot, axis=0, keepdims=True)
        cnt_sc[...] = jnp.where(i == 0, csum, cnt_sc[...] + csum)

    @pl.when(i == N_RT - 1)
    def _offsets():
        padded = ((cnt_sc[...].astype(jnp.int32) + (PAD - 1))
                  & ~(PAD - 1)).astype(jnp.float32)
        row_i = lax.broadcasted_iota(jnp.int32, (N_EXPERTS, 2 * N_EXPERTS), 0)
        col_i = lax.broadcasted_iota(jnp.int32, (N_EXPERTS, 2 * N_EXPERTS), 1)
        tri = (row_i < col_i).astype(jnp.float32)
        offs = lax.dot_general(
            padded, tri, (((1,), (0,)), ((), ())),
            preferred_element_type=jnp.float32)
        offs_ref[...] = offs.astype(jnp.int32)
        base_sc[...] = offs[:, :N_EXPERTS]

    @pl.when(i >= N_RT)
    def _pass2():
        t = i - N_RT
        ri = lax.broadcasted_iota(jnp.int32, (RBLK, RBLK), 0)
        ci = lax.broadcasted_iota(jnp.int32, (RBLK, RBLK), 1)
        tri_b = (ci < ri).astype(jnp.float32)
        oh_t = oh_sc[pl.ds(t * RBLK, RBLK), :]
        within = lax.dot_general(
            tri_b, oh_t, (((1,), (0,)), ((), ())),
            preferred_element_type=jnp.float32)
        base = base_sc[...]
        pos_t = jnp.sum(oh_t * (within + base), axis=1, keepdims=True)
        pos_ref[...] = pos_t.astype(jnp.int32)
        base_sc[...] = base + jnp.sum(oh_t, axis=0, keepdims=True)


def _router(x, Wr, br):
    last = N_RT - 1
    return pl.pallas_call(
        _router_body,
        grid=(2 * N_RT,),
        in_specs=[
            pl.BlockSpec((RBLK, D_MODEL), lambda i: (jnp.minimum(i, last), 0)),
            pl.BlockSpec((N_EXPERTS, D_MODEL), lambda i: (0, 0)),
            pl.BlockSpec((1, N_EXPERTS), lambda i: (0, 0)),
        ],
        out_specs=[
            pl.BlockSpec((RBLK, N_EXPERTS), lambda i: (jnp.minimum(i, last), 0)),
            pl.BlockSpec((RBLK, 1), lambda i: (jnp.minimum(i, last), 0)),
            pl.BlockSpec((RBLK, 1), lambda i: (jnp.maximum(i - N_RT, 0), 0)),
            pl.BlockSpec((1, 2 * N_EXPERTS), lambda i: (0, 0)),
        ],
        out_shape=[
            jax.ShapeDtypeStruct((N_TOKENS, N_EXPERTS), jnp.float32),
            jax.ShapeDtypeStruct((N_TOKENS, 1), jnp.float32),
            jax.ShapeDtypeStruct((N_TOKENS, 1), jnp.int32),
            jax.ShapeDtypeStruct((1, 2 * N_EXPERTS), jnp.int32),
        ],
        scratch_shapes=[
            pltpu.VMEM((N_TOKENS, N_EXPERTS), jnp.float32),
            pltpu.VMEM((1, N_EXPERTS), jnp.float32),
            pltpu.VMEM((1, N_EXPERTS), jnp.float32),
        ],
        interpret=_INTERPRET,
    )(x, Wr, br.reshape(1, N_EXPERTS))


def _dispatch_body(pos_hbm, w_hbm, x_hbm, ws_hbm, xs_hbm,
                   pos_v, w_v, xrows_v, wbuf_v, semg, sems):
    wid = lax.axis_index("s") * 2 + lax.axis_index("c")
    base = pl.multiple_of(wid * TOK_PER_W, TOK_PER_W)
    pltpu.async_copy(pos_hbm.at[pl.ds(base, TOK_PER_W)], pos_v, semg)
    pltpu.async_copy(w_hbm.at[pl.ds(base, TOK_PER_W)], w_v, semg)
    pltpu.async_copy(x_hbm.at[pl.ds(base, TOK_PER_W)], xrows_v, semg)
    pltpu.make_async_copy(pos_hbm.at[pl.ds(base, TOK_PER_W)], pos_v,
                          semg).wait()
    pltpu.make_async_copy(w_hbm.at[pl.ds(base, TOK_PER_W)], w_v, semg).wait()
    pltpu.make_async_copy(x_hbm.at[pl.ds(base, TOK_PER_W)], xrows_v,
                          semg).wait()
    lanes = lax.iota(jnp.int32, 16)
    zeros = jnp.zeros((16,), jnp.int32)
    for j in range(TOK_PER_W // 16):
        plsc.store_scatter(wbuf_v, [lanes + j * 16, zeros],
                           w_v[pl.ds(j * 16, 16)])
    for j in range(TOK_PER_W // 16):
        posvec = pos_v[pl.ds(j * 16, 16)]
        pltpu.async_copy(xrows_v.at[pl.ds(j * 16, 16)],
                         xs_hbm.at[posvec], sems)
        pltpu.async_copy(wbuf_v.at[pl.ds(j * 16, 16)],
                         ws_hbm.at[posvec], sems)
    for j in range(TOK_PER_W // 16):
        pltpu.make_async_copy(xs_hbm.at[pl.ds(0, 16)],
                              xrows_v.at[pl.ds(0, 16)], sems).wait()
        pltpu.make_async_copy(ws_hbm.at[pl.ds(0, 16)],
                              wbuf_v.at[pl.ds(0, 16)], sems).wait()


def _dispatch(pos, w, x):
    mesh = plsc.VectorSubcoreMesh(core_axis_name="c", subcore_axis_name="s")
    f = pl.kernel(
        _dispatch_body,
        out_type=[
            jax.ShapeDtypeStruct((N_BUF, 128), jnp.float32),
            jax.ShapeDtypeStruct((N_BUF, D_MODEL), jnp.float32),
        ],
        mesh=mesh,
        compiler_params=pltpu.CompilerParams(needs_layout_passes=False),
        scratch_types=[
            pltpu.VMEM((TOK_PER_W,), jnp.int32),
            pltpu.VMEM((TOK_PER_W,), jnp.float32),
            pltpu.VMEM((TOK_PER_W, D_MODEL), jnp.float32),
            pltpu.VMEM((TOK_PER_W, 128), jnp.float32),
            pltpu.SemaphoreType.DMA,
            pltpu.SemaphoreType.DMA,
        ],
    )
    return f(pos, w, x)


XCH = N_BUF // 8


def _expert_body(offs_ref, xs_hbm, ws_ref, w_ref, b_ref, ys_ref,
                 xbuf, semx, dr_ref):
    e = pl.program_id(0)
    start = offs_ref[e]
    stop = offs_ref[e + 1]

    @pl.when(e == 0)
    def _():
        dr_ref[0] = 0
        for j in range(8):
            pltpu.make_async_copy(
                xs_hbm.at[pl.ds(j * XCH, XCH)],
                xbuf.at[pl.ds(j * XCH, XCH)], semx).start()

    need = jnp.where(e == N_EXPERTS - 1, 8,
                     jnp.minimum((stop + ROW_TILE - PAD + XCH - 1) // XCH, 8))
    done = dr_ref[0]

    def drain(r, carry):
        pltpu.make_async_copy(xs_hbm.at[pl.ds(0, XCH)],
                              xbuf.at[pl.ds(0, XCH)], semx).wait()
        return carry

    lax.fori_loop(0, need - done, drain, 0)
    dr_ref[0] = jnp.maximum(need, done)

    n_tiles = (stop - start + ROW_TILE - 1) // ROW_TILE

    def tile(t, carry):
        s = pl.multiple_of(start + t * ROW_TILE, PAD)
        xt = xbuf[pl.ds(s, ROW_TILE), :]
        y = lax.dot_general(
            xt, w_ref[0], (((1,), (1,)), ((), ())),
            preferred_element_type=jnp.float32)
        y = (y + b_ref[0]) * ws_ref[pl.ds(s, ROW_TILE), 0:1]
        ys_ref[pl.ds(s, ROW_TILE), :] = y
        return carry

    lax.fori_loop(0, n_tiles, tile, 0)


def _expert_matmul(offsets, x_sorted, w_sorted, W, b):
    grid_spec = pltpu.PrefetchScalarGridSpec(
        num_scalar_prefetch=1,
        grid=(N_EXPERTS,),
        in_specs=[
            pl.BlockSpec(memory_space=pl.ANY),
            pl.BlockSpec((N_BUF, 128), lambda e, offs: (0, 0)),
            pl.BlockSpec((1, D_MODEL, D_MODEL), lambda e, offs: (e, 0, 0)),
            pl.BlockSpec((1, 1, D_MODEL), lambda e, offs: (e, 0, 0)),
        ],
        out_specs=pl.BlockSpec((N_BUF, D_MODEL), lambda e, offs: (0, 0)),
        scratch_shapes=[
            pltpu.VMEM((N_BUF, D_MODEL), jnp.float32),
            pltpu.SemaphoreType.DMA,
            pltpu.SMEM((1,), jnp.int32),
        ],
    )
    return pl.pallas_call(
        _expert_body,
        grid_spec=grid_spec,
        out_shape=jax.ShapeDtypeStruct((N_BUF, D_MODEL), jnp.float32),
        interpret=_INTERPRET,
    )(offsets, x_sorted, w_sorted, W, b.reshape(N_EXPERTS, 1, D_MODEL))


def _combine_body(ys_hbm, pos_hbm, out_hbm, pos_v, yrows_v, semg):
    wid = lax.axis_index("s") * 2 + lax.axis_index("c")
    base = pl.multiple_of(wid * TOK_PER_W, TOK_PER_W)
    pltpu.async_copy(pos_hbm.at[pl.ds(base, TOK_PER_W)], pos_v, semg)
    pltpu.make_async_copy(pos_hbm.at[pl.ds(base, TOK_PER_W)], pos_v,
                          semg).wait()
    for j in range(TOK_PER_W // 16):
        posvec = pos_v[pl.ds(j * 16, 16)]
        pltpu.async_copy(ys_hbm.at[posvec],
                         yrows_v.at[pl.ds(j * 16, 16)], semg)
    for j in range(TOK_PER_W // 16):
        pltpu.make_async_copy(ys_hbm.at[pl.ds(0, 16)],
                              yrows_v.at[pl.ds(0, 16)], semg).wait()
    pltpu.sync_copy(yrows_v, out_hbm.at[pl.ds(base, TOK_PER_W)])


def _combine(y_sorted, pos):
    mesh = plsc.VectorSubcoreMesh(core_axis_name="c", subcore_axis_name="s")
    f = pl.kernel(
        _combine_body,
        out_type=jax.ShapeDtypeStruct((N_TOKENS, D_MODEL), jnp.float32),
        mesh=mesh,
        compiler_params=pltpu.CompilerParams(needs_layout_passes=False),
        scratch_types=[
            pltpu.VMEM((TOK_PER_W,), jnp.int32),
            pltpu.VMEM((TOK_PER_W, D_MODEL), jnp.float32),
            pltpu.SemaphoreType.DMA,
        ],
    )
    return f(y_sorted, pos)


def kernel(x, W, b, Wr, br):
    logits, w, pos, offs = _router(x, Wr, br)
    pos_flat = pos.reshape(N_TOKENS)
    offs_flat = offs.reshape(2 * N_EXPERTS)
    w_sorted, x_sorted = _dispatch(pos_flat, w.reshape(N_TOKENS), x)
    y_sorted = _expert_matmul(offs_flat, x_sorted, w_sorted, W, b)
    output = _combine(y_sorted, pos_flat)
    return (output, logits)


# speedup vs baseline: 1.1156x; 1.1156x over previous
import jax
import jax.numpy as jnp
from jax import lax
from jax.experimental import pallas as pl
from jax.experimental.pallas import tpu as pltpu
from jax.experimental.pallas import tpu_sc as plsc

D_MODEL = 768
N_EXPERTS = 64
N_TOKENS = 2048
ROW_TILE = 64
N_BUF = 2560
PAD = 8
TOK_PER_W = 64
RBLK = 128


def _router_body(x_ref, wr_ref, br_ref, logits_ref, w_ref, pos_ref, offs_ref):
    x = x_ref[...]
    logits = lax.dot_general(
        x, wr_ref[...], (((1,), (1,)), ((), ())),
        preferred_element_type=jnp.float32) + br_ref[...]
    logits_ref[...] = logits
    m = jnp.max(logits, axis=1, keepdims=True)
    p = jnp.exp(logits - m)
    s = jnp.sum(p, axis=1, keepdims=True)
    w_ref[...] = 1.0 / s
    lane = lax.broadcasted_iota(jnp.int32, (N_TOKENS, N_EXPERTS), 1)
    cand = jnp.where(logits == m, lane, N_EXPERTS)
    idx = jnp.min(cand, axis=1, keepdims=True)
    onehot = (lane == idx).astype(jnp.float32)
    cnt = jnp.sum(onehot, axis=0, keepdims=True)
    padded = ((cnt.astype(jnp.int32) + (PAD - 1)) & ~(PAD - 1)).astype(
        jnp.float32)
    row_i = lax.broadcasted_iota(jnp.int32, (N_EXPERTS, 2 * N_EXPERTS), 0)
    col_i = lax.broadcasted_iota(jnp.int32, (N_EXPERTS, 2 * N_EXPERTS), 1)
    tri = (row_i < col_i).astype(jnp.float32)
    offs = lax.dot_general(
        padded, tri, (((1,), (0,)), ((), ())),
        preferred_element_type=jnp.float32)
    offs_ref[...] = offs.astype(jnp.int32)
    ri = lax.broadcasted_iota(jnp.int32, (RBLK, RBLK), 0)
    ci = lax.broadcasted_iota(jnp.int32, (RBLK, RBLK), 1)
    tri_b = (ci < ri).astype(jnp.float32)
    offs64 = offs[:, :N_EXPERTS]
    base = jnp.zeros((1, N_EXPERTS), jnp.float32)
    for t in range(N_TOKENS // RBLK):
        oh_t = onehot[t * RBLK:(t + 1) * RBLK, :]
        within = lax.dot_general(
            tri_b, oh_t, (((1,), (0,)), ((), ())),
            preferred_element_type=jnp.float32)
        pos_t = jnp.sum(oh_t * (within + base + offs64), axis=1,
                        keepdims=True)
        pos_ref[t * RBLK:(t + 1) * RBLK, :] = pos_t.astype(jnp.int32)
        base = base + jnp.sum(oh_t, axis=0, keepdims=True)


def _router(x, Wr, br):
    return pl.pallas_call(
        _router_body,
        out_shape=[
            jax.ShapeDtypeStruct((N_TOKENS, N_EXPERTS), jnp.float32),
            jax.ShapeDtypeStruct((N_TOKENS, 1), jnp.float32),
            jax.ShapeDtypeStruct((N_TOKENS, 1), jnp.int32),
            jax.ShapeDtypeStruct((1, 2 * N_EXPERTS), jnp.int32),
        ],
    )(x, Wr, br.reshape(1, N_EXPERTS))


def _dispatch_body(pos_hbm, w_hbm, x_hbm, ws_hbm, xs_hbm,
                   pos_v, w_v, xrows_v, wbuf_v, semg, sems):
    wid = lax.axis_index("s") * 2 + lax.axis_index("c")
    base = pl.multiple_of(wid * TOK_PER_W, TOK_PER_W)
    pltpu.async_copy(pos_hbm.at[pl.ds(base, TOK_PER_W)], pos_v, semg)
    pltpu.async_copy(w_hbm.at[pl.ds(base, TOK_PER_W)], w_v, semg)
    pltpu.async_copy(x_hbm.at[pl.ds(base, TOK_PER_W)], xrows_v, semg)
    pltpu.make_async_copy(pos_hbm.at[pl.ds(base, TOK_PER_W)], pos_v,
                          semg).wait()
    pltpu.make_async_copy(w_hbm.at[pl.ds(base, TOK_PER_W)], w_v, semg).wait()
    pltpu.make_async_copy(x_hbm.at[pl.ds(base, TOK_PER_W)], xrows_v,
                          semg).wait()
    lanes = lax.iota(jnp.int32, 16)
    zeros = jnp.zeros((16,), jnp.int32)
    for j in range(TOK_PER_W // 16):
        plsc.store_scatter(wbuf_v, [lanes + j * 16, zeros],
                           w_v[pl.ds(j * 16, 16)])
    for j in range(TOK_PER_W // 16):
        posvec = pos_v[pl.ds(j * 16, 16)]
        pltpu.async_copy(xrows_v.at[pl.ds(j * 16, 16)],
                         xs_hbm.at[posvec], sems)
        pltpu.async_copy(wbuf_v.at[pl.ds(j * 16, 16)],
                         ws_hbm.at[posvec], sems)
    for j in range(TOK_PER_W // 16):
        pltpu.make_async_copy(xs_hbm.at[pl.ds(0, 16)],
                              xrows_v.at[pl.ds(0, 16)], sems).wait()
        pltpu.make_async_copy(ws_hbm.at[pl.ds(0, 16)],
                              wbuf_v.at[pl.ds(0, 16)], sems).wait()


def _dispatch(pos, w, x):
    mesh = plsc.VectorSubcoreMesh(core_axis_name="c", subcore_axis_name="s")
    f = pl.kernel(
        _dispatch_body,
        out_type=[
            jax.ShapeDtypeStruct((N_BUF, 128), jnp.float32),
            jax.ShapeDtypeStruct((N_BUF, D_MODEL), jnp.float32),
        ],
        mesh=mesh,
        compiler_params=pltpu.CompilerParams(needs_layout_passes=False),
        scratch_types=[
            pltpu.VMEM((TOK_PER_W,), jnp.int32),
            pltpu.VMEM((TOK_PER_W,), jnp.float32),
            pltpu.VMEM((TOK_PER_W, D_MODEL), jnp.float32),
            pltpu.VMEM((TOK_PER_W, 128), jnp.float32),
            pltpu.SemaphoreType.DMA,
            pltpu.SemaphoreType.DMA,
        ],
    )
    return f(pos, w, x)


XCH = N_BUF // 8


def _expert_body(offs_ref, xs_hbm, ws_ref, w_ref, b_ref, ys_ref,
                 xbuf, semx, dr_ref):
    e = pl.program_id(0)
    start = offs_ref[e]
    stop = offs_ref[e + 1]

    @pl.when(e == 0)
    def _():
        dr_ref[0] = 0
        for j in range(8):
            pltpu.make_async_copy(
                xs_hbm.at[pl.ds(j * XCH, XCH)],
                xbuf.at[pl.ds(j * XCH, XCH)], semx).start()

    need = jnp.where(e == N_EXPERTS - 1, 8,
                     jnp.minimum((stop + ROW_TILE - PAD + XCH - 1) // XCH, 8))
    done = dr_ref[0]

    def drain(r, carry):
        pltpu.make_async_copy(xs_hbm.at[pl.ds(0, XCH)],
                              xbuf.at[pl.ds(0, XCH)], semx).wait()
        return carry

    lax.fori_loop(0, need - done, drain, 0)
    dr_ref[0] = jnp.maximum(need, done)

    n_tiles = (stop - start + ROW_TILE - 1) // ROW_TILE

    def tile(t, carry):
        s = pl.multiple_of(start + t * ROW_TILE, PAD)
        xt = xbuf[pl.ds(s, ROW_TILE), :]
        y = lax.dot_general(
            xt, w_ref[0], (((1,), (1,)), ((), ())),
            preferred_element_type=jnp.float32)
        y = (y + b_ref[0]) * ws_ref[pl.ds(s, ROW_TILE), 0:1]
        ys_ref[pl.ds(s, ROW_TILE), :] = y
        return carry

    lax.fori_loop(0, n_tiles, tile, 0)


def _expert_matmul(offsets, x_sorted, w_sorted, W, b):
    grid_spec = pltpu.PrefetchScalarGridSpec(
        num_scalar_prefetch=1,
        grid=(N_EXPERTS,),
        in_specs=[
            pl.BlockSpec(memory_space=pl.ANY),
            pl.BlockSpec((N_BUF, 128), lambda e, offs: (0, 0)),
            pl.BlockSpec((1, D_MODEL, D_MODEL), lambda e, offs: (e, 0, 0)),
            pl.BlockSpec((1, 1, D_MODEL), lambda e, offs: (e, 0, 0)),
        ],
        out_specs=pl.BlockSpec((N_BUF, D_MODEL), lambda e, offs: (0, 0)),
        scratch_shapes=[
            pltpu.VMEM((N_BUF, D_MODEL), jnp.float32),
            pltpu.SemaphoreType.DMA,
            pltpu.SMEM((1,), jnp.int32),
        ],
    )
    return pl.pallas_call(
        _expert_body,
        grid_spec=grid_spec,
        out_shape=jax.ShapeDtypeStruct((N_BUF, D_MODEL), jnp.float32),
    )(offsets, x_sorted, w_sorted, W, b.reshape(N_EXPERTS, 1, D_MODEL))


def _combine_body(ys_hbm, pos_hbm, out_hbm, pos_v, yrows_v, semg):
    wid = lax.axis_index("s") * 2 + lax.axis_index("c")
    base = pl.multiple_of(wid * TOK_PER_W, TOK_PER_W)
    pltpu.async_copy(pos_hbm.at[pl.ds(base, TOK_PER_W)], pos_v, semg)
    pltpu.make_async_copy(pos_hbm.at[pl.ds(base, TOK_PER_W)], pos_v,
                          semg).wait()
    for j in range(TOK_PER_W // 16):
        posvec = pos_v[pl.ds(j * 16, 16)]
        pltpu.async_copy(ys_hbm.at[posvec],
                         yrows_v.at[pl.ds(j * 16, 16)], semg)
    for j in range(TOK_PER_W // 16):
        pltpu.make_async_copy(ys_hbm.at[pl.ds(0, 16)],
                              yrows_v.at[pl.ds(0, 16)], semg).wait()
    pltpu.sync_copy(yrows_v, out_hbm.at[pl.ds(base, TOK_PER_W)])


def _combine(y_sorted, pos):
    mesh = plsc.VectorSubcoreMesh(core_axis_name="c", subcore_axis_name="s")
    f = pl.kernel(
        _combine_body,
        out_type=jax.ShapeDtypeStruct((N_TOKENS, D_MODEL), jnp.float32),
        mesh=mesh,
        compiler_params=pltpu.CompilerParams(needs_layout_passes=False),
        scratch_types=[
            pltpu.VMEM((TOK_PER_W,), jnp.int32),
            pltpu.VMEM((TOK_PER_W, D_MODEL), jnp.float32),
            pltpu.SemaphoreType.DMA,
        ],
    )
    return f(y_sorted, pos)


def kernel(x, W, b, Wr, br):
    logits, w, pos, offs = _router(x, Wr, br)
    pos_flat = pos.reshape(N_TOKENS)
    offs_flat = offs.reshape(2 * N_EXPERTS)
    w_sorted, x_sorted = _dispatch(pos_flat, w.reshape(N_TOKENS), x)
    y_sorted = _expert_matmul(offs_flat, x_sorted, w_sorted, W, b)
    output = _combine(y_sorted, pos_flat)
    return (output, logits)
